# baseline (speedup 1.0000x reference)
import functools

import jax
import jax.numpy as jnp
from jax import lax
from jax.experimental import pallas as pl
from jax.experimental.pallas import tpu as pltpu
from jax.experimental.pallas import tpu_sc as plsc

ENT_TOT = 1000000
REL_TOT = 1000
B = 16384
DIM = 64
PDIM = 128
NC = 2
NS = 16
NW = NC * NS
ROWS_PER_W = B // NW
CHUNK = 128
NCHUNK = ROWS_PER_W // CHUNK
GROUPS = CHUNK // 16
TCW = 512
ENT_NB = (ENT_TOT + TCW - 1) // TCW
ENT_HALF = (ENT_NB // 2) * TCW
REL_NB = 2
REL_HALF = TCW

_F32 = jnp.float32
_MAGIC = 0x5F3759DF


def _rsqrt(x):
    i = plsc.bitcast(x, jnp.int32)
    y = plsc.bitcast(jnp.int32(_MAGIC) - (i >> 1), _F32)
    for _ in range(3):
        y = y * (_F32(1.5) - _F32(0.5) * x * y * y)
    return y


def _sqrt(x):
    return x * _rsqrt(jnp.maximum(x, _F32(1e-30)))


def _xpose_body(x1_ref, x2_ref, o_ref):
    y1 = jnp.transpose(x1_ref[...])
    y2 = jnp.transpose(x2_ref[...])
    o_ref[...] = jnp.concatenate([y1, y2], axis=1)


def _pack_table(table_t, nblocks):
    nb2 = nblocks // 2
    return pl.pallas_call(
        _xpose_body,
        grid=(nb2,),
        in_specs=[pl.BlockSpec((DIM, TCW), lambda i: (0, i)),
                  pl.BlockSpec((DIM, TCW), lambda i, _nb2=nb2: (0, i + _nb2))],
        out_specs=pl.BlockSpec((TCW, PDIM), lambda i: (i, 0)),
        out_shape=jax.ShapeDtypeStruct((nb2 * TCW, PDIM), _F32),
    )(table_t, table_t)


def _sc_body(head_hbm, rel_hbm, tail_hbm, negv_hbm, ent_hbm, relemb_hbm,
             pos_out, neg_out, dist_out,
             idx_h, idx_r, idx_t, idx_n,
             half_h, half_r, half_t, half_n,
             h_buf, r_buf, t_buf, n_buf,
             inv_h, inv_t, inv_n,
             pos_b, neg1_b, neg2_b, dist_b, sem):
    cid = lax.axis_index("c")
    sid = lax.axis_index("s")
    wid = sid * NC + cid
    base = wid * ROWS_PER_W
    lane = lax.broadcasted_iota(jnp.int32, (16,), 0)
    zero = jnp.zeros((16,), _F32)

    def chunk_body(c, dist_acc):
        cbase = base + c * CHUNK
        pltpu.sync_copy(head_hbm.at[pl.ds(cbase, CHUNK)], idx_h.at[c])
        pltpu.sync_copy(rel_hbm.at[pl.ds(cbase, CHUNK)], idx_r.at[c])
        pltpu.sync_copy(tail_hbm.at[pl.ds(cbase, CHUNK)], idx_t.at[c])
        pltpu.sync_copy(negv_hbm.at[pl.ds(cbase, CHUNK)], idx_n.at[c])

        def split(g, carry):
            gs = pl.ds(g * 16, 16)
            hv, rv = idx_h[c, gs], idx_r[c, gs]
            tv, nv = idx_t[c, gs], idx_n[c, gs]
            half_h[gs] = jnp.where(hv >= ENT_HALF, hv - ENT_HALF, hv)
            half_r[gs] = jnp.where(rv >= REL_HALF, rv - REL_HALF, rv)
            half_t[gs] = jnp.where(tv >= ENT_HALF, tv - ENT_HALF, tv)
            half_n[gs] = jnp.where(nv >= ENT_HALF, nv - ENT_HALF, nv)
            return carry

        lax.fori_loop(0, GROUPS, split, 0)

        cp_h = pltpu.async_copy(ent_hbm.at[half_h], h_buf, sem)
        cp_r = pltpu.async_copy(relemb_hbm.at[half_r], r_buf, sem)
        cp_t = pltpu.async_copy(ent_hbm.at[half_t], t_buf, sem)
        cp_n = pltpu.async_copy(ent_hbm.at[half_n], n_buf, sem)
        cp_h.wait()
        cp_r.wait()
        cp_t.wait()
        cp_n.wait()

        def pass1(g, carry):
            gs = pl.ds(g * 16, 16)
            ph = jnp.where(idx_h[c, gs] >= ENT_HALF, 64, 0)
            pt = jnp.where(idx_t[c, gs] >= ENT_HALF, 64, 0)
            pn = jnp.where(idx_n[c, gs] >= ENT_HALF, 64, 0)
            sh_v, st_v, sn_v = zero, zero, zero
            for i in range(16):
                row = g * 16 + i

                def rowsq(buf, pv):
                    bb = pv[i]
                    a = buf[row, pl.ds(bb, 16)]
                    b = buf[row, pl.ds(bb + 16, 16)]
                    cc = buf[row, pl.ds(bb + 32, 16)]
                    d = buf[row, pl.ds(bb + 48, 16)]
                    return jnp.sum(a * a + b * b + cc * cc + d * d)

                sh_v = jnp.where(lane == i, rowsq(h_buf, ph), sh_v)
                st_v = jnp.where(lane == i, rowsq(t_buf, pt), st_v)
                sn_v = jnp.where(lane == i, rowsq(n_buf, pn), sn_v)
            inv_h[gs] = _rsqrt(jnp.maximum(sh_v, _F32(1e-24)))
            inv_t[gs] = _rsqrt(jnp.maximum(st_v, _F32(1e-24)))
            inv_n[gs] = _rsqrt(jnp.maximum(sn_v, _F32(1e-24)))
            return carry

        lax.fori_loop(0, GROUPS, pass1, 0)

        def pass2(g, d_acc):
            gs = pl.ds(g * 16, 16)
            ph = jnp.where(idx_h[c, gs] >= ENT_HALF, 64, 0)
            pr = jnp.where(idx_r[c, gs] >= REL_HALF, 64, 0)
            pt = jnp.where(idx_t[c, gs] >= ENT_HALF, 64, 0)
            pn = jnp.where(idx_n[c, gs] >= ENT_HALF, 64, 0)
            ihv = inv_h[gs]
            itv = inv_t[gs]
            iqv = inv_n[gs]
            sp_v, s1_v, s2_v, sd_v = zero, zero, zero, zero
            for i in range(16):
                row = g * 16 + i
                ih = ihv[i]
                it = itv[i]
                iq = iqv[i]
                bh, br, bt, bn = ph[i], pr[i], pt[i], pn[i]
                acc_p = acc_1 = acc_2 = acc_d = None
                for k in range(4):
                    o = k * 16
                    hk = h_buf[row, pl.ds(bh + o, 16)]
                    rk = r_buf[row, pl.ds(br + o, 16)]
                    tk = t_buf[row, pl.ds(bt + o, 16)]
                    nk = n_buf[row, pl.ds(bn + o, 16)]
                    hn = hk * ih
                    tn = tk * it
                    nn = nk * iq
                    cc = hn + rk
                    bb = rk - tn
                    pv = cc - tn
                    n1 = bb + nn
                    n2 = cc - nn
                    dv = hk - tk
                    if acc_p is None:
                        acc_p, acc_1 = pv * pv, n1 * n1
                        acc_2, acc_d = n2 * n2, dv * dv
                    else:
                        acc_p = acc_p + pv * pv
                        acc_1 = acc_1 + n1 * n1
                        acc_2 = acc_2 + n2 * n2
                        acc_d = acc_d + dv * dv
                sp_v = jnp.where(lane == i, jnp.sum(acc_p), sp_v)
                s1_v = jnp.where(lane == i, jnp.sum(acc_1), s1_v)
                s2_v = jnp.where(lane == i, jnp.sum(acc_2), s2_v)
                sd_v = jnp.where(lane == i, jnp.sum(acc_d), sd_v)
            pos_b[gs] = -_sqrt(sp_v)
            neg1_b[gs] = -_sqrt(s1_v)
            neg2_b[gs] = -_sqrt(s2_v)
            return d_acc + _sqrt(sd_v)

        dist_acc = lax.fori_loop(0, GROUPS, pass2, dist_acc)

        pltpu.sync_copy(pos_b, pos_out.at[pl.ds(cbase, CHUNK)])
        pltpu.sync_copy(pos_b, pos_out.at[pl.ds(B + cbase, CHUNK)])
        pltpu.sync_copy(neg1_b, neg_out.at[pl.ds(cbase, CHUNK)])
        pltpu.sync_copy(neg2_b, neg_out.at[pl.ds(B + cbase, CHUNK)])
        return dist_acc

    dist_acc = lax.fori_loop(0, NCHUNK, chunk_body, zero)
    dist_b[...] = dist_acc
    pltpu.sync_copy(dist_b, dist_out.at[wid])


@functools.partial(jax.jit, static_argnames=())
def _sc_call(batch_head, batch_rel, batch_tail, batch_negative, ent2, rel2):
    mesh = plsc.VectorSubcoreMesh(core_axis_name="c", subcore_axis_name="s",
                                  num_cores=NC, num_subcores=NS)
    f = pl.kernel(
        _sc_body,
        out_type=(
            jax.ShapeDtypeStruct((2 * B,), _F32),
            jax.ShapeDtypeStruct((2 * B,), _F32),
            jax.ShapeDtypeStruct((NW, 16), _F32),
        ),
        mesh=mesh,
        compiler_params=pltpu.CompilerParams(needs_layout_passes=False),
        scratch_types=[
            pltpu.VMEM((NCHUNK, CHUNK), jnp.int32),
            pltpu.VMEM((NCHUNK, CHUNK), jnp.int32),
            pltpu.VMEM((NCHUNK, CHUNK), jnp.int32),
            pltpu.VMEM((NCHUNK, CHUNK), jnp.int32),
            pltpu.VMEM((CHUNK,), jnp.int32),
            pltpu.VMEM((CHUNK,), jnp.int32),
            pltpu.VMEM((CHUNK,), jnp.int32),
            pltpu.VMEM((CHUNK,), jnp.int32),
            pltpu.VMEM((CHUNK, PDIM), _F32),
            pltpu.VMEM((CHUNK, PDIM), _F32),
            pltpu.VMEM((CHUNK, PDIM), _F32),
            pltpu.VMEM((CHUNK, PDIM), _F32),
            pltpu.VMEM((CHUNK,), _F32),
            pltpu.VMEM((CHUNK,), _F32),
            pltpu.VMEM((CHUNK,), _F32),
            pltpu.VMEM((CHUNK,), _F32),
            pltpu.VMEM((CHUNK,), _F32),
            pltpu.VMEM((CHUNK,), _F32),
            pltpu.VMEM((16,), _F32),
            pltpu.SemaphoreType.DMA,
        ],
    )
    return f(batch_head, batch_rel, batch_tail, batch_negative, ent2, rel2)


def kernel(batch_head, batch_rel, batch_tail, batch_negative, ent_emb, rel_emb):
    ent2 = _pack_table(ent_emb.T, ENT_NB)
    rel2 = _pack_table(rel_emb.T, REL_NB)
    pos, neg, dist_parts = _sc_call(batch_head, batch_rel, batch_tail,
                                    batch_negative, ent2, rel2)
    return pos, neg, jnp.sum(dist_parts)
